# baseline (speedup 1.0000x reference)
import functools

import jax
import jax.numpy as jnp
from jax import lax
from jax.experimental import pallas as pl
from jax.experimental.pallas import tpu as pltpu
from jax.experimental.pallas import tpu_sc as plsc

_NC = 2
_NS = 16
_NW = _NC * _NS
_BB = 128
_NBUF = 2


@functools.lru_cache(maxsize=None)
def _make_gather(Bt, L, D):
    assert D == 64 and Bt == _NW * _BB
    mesh = plsc.VectorSubcoreMesh(
        core_axis_name="c", subcore_axis_name="s",
        num_cores=_NC, num_subcores=_NS)

    scratch = (
        [pltpu.VMEM((L, _BB), jnp.int32),
         pltpu.VMEM((L, _BB), jnp.int32)]
        + [pltpu.VMEM((_BB, 2 * D), jnp.float32) for _ in range(_NBUF)]
        + [pltpu.VMEM((D, _BB), jnp.float32) for _ in range(_NBUF)]
        + [pltpu.SemaphoreType.DMA for _ in range(2 * _NBUF)]
    )

    @functools.partial(
        pl.kernel,
        out_type=jax.ShapeDtypeStruct((L, D, Bt), jnp.float32),
        mesh=mesh,
        scratch_types=scratch,
        compiler_params=pltpu.CompilerParams(
            use_tc_tiling_on_sc=True, needs_layout_passes=False),
    )
    def gather_kernel(idxh_hbm, colx_hbm, table_hbm, out_hbm, idxh_v,
                      colx_v, *rest):
        pair = rest[:_NBUF]
        tbuf = rest[_NBUF:2 * _NBUF]
        gsem = rest[2 * _NBUF:3 * _NBUF]
        wsem = rest[3 * _NBUF:]
        wid = lax.axis_index("s") * _NC + lax.axis_index("c")
        b0 = wid * _BB

        pltpu.sync_copy(idxh_hbm.at[wid], idxh_v)
        pltpu.sync_copy(colx_hbm.at[wid], colx_v)

        for b in range(_NBUF):
            pltpu.async_copy(table_hbm.at[idxh_v.at[b]], pair[b], gsem[b])

        lanes = lax.iota(jnp.int32, 16)

        @pl.loop(0, L, step=_NBUF)
        def _(g):
            for b in range(_NBUF):
                l = g + b
                pltpu.make_async_copy(
                    table_hbm.at[idxh_v.at[l]], pair[b], gsem[b]).wait()

                @pl.when(l >= _NBUF)
                def _():
                    pltpu.make_async_copy(
                        tbuf[b],
                        out_hbm.at[l - _NBUF, :, pl.ds(b0, _BB)],
                        wsem[b]).wait()

                groups = [(k0 + lanes, colx_v[l, pl.ds(k0, 16)])
                          for k0 in range(0, _BB, 16)]

                @pl.loop(0, D, step=8)
                def _(d):
                    for d0 in range(0, 8, 4):
                        vals = [plsc.load_gather(
                                    pair[b], [rowv, colb + (d + (d0 + u))])
                                for u in range(4)
                                for (rowv, colb) in groups]
                        vi = 0
                        for u in range(4):
                            for gi in range(len(groups)):
                                tbuf[b][d + (d0 + u),
                                        pl.ds(gi * 16, 16)] = vals[vi]
                                vi += 1

                pltpu.async_copy(
                    tbuf[b], out_hbm.at[l, :, pl.ds(b0, _BB)], wsem[b])

                @pl.when(l + _NBUF < L)
                def _():
                    pltpu.async_copy(
                        table_hbm.at[idxh_v.at[l + _NBUF]], pair[b], gsem[b])

        for b in range(_NBUF):
            l = L - _NBUF + b
            pltpu.make_async_copy(
                tbuf[b], out_hbm.at[l, :, pl.ds(b0, _BB)], wsem[b]).wait()

    return gather_kernel


def kernel(text, embedding_weight):
    Bt, L = text.shape
    V, D = embedding_weight.shape
    tt = jnp.transpose(text.astype(jnp.int32), (1, 0))
    tt = jnp.transpose(tt.reshape(L, _NW, _BB), (1, 0, 2))
    idxh = tt >> 1
    colx = (tt & 1) * D
    table2 = embedding_weight.reshape(V // 2, 2 * D)
    out = _make_gather(Bt, L, D)(idxh, colx, table2)
    return jnp.transpose(out[None], (3, 0, 1, 2))


# speedup vs baseline: 1.2795x; 1.2795x over previous
import functools

import jax
import jax.numpy as jnp
from jax import lax
from jax.experimental import pallas as pl
from jax.experimental.pallas import tpu as pltpu
from jax.experimental.pallas import tpu_sc as plsc

_NC = 2
_NS = 16
_NW = _NC * _NS
_CHUNK = 512
_NBUF = 2


@functools.lru_cache(maxsize=None)
def _make_gather(B, D):
    b_per_w = B // _NW
    nchunks = b_per_w // _CHUNK
    mesh = plsc.VectorSubcoreMesh(
        core_axis_name="c", subcore_axis_name="s",
        num_cores=_NC, num_subcores=_NS)

    scratch = (
        [pltpu.VMEM((nchunks, _CHUNK), jnp.int32)]
        + [pltpu.VMEM((_CHUNK, D), jnp.float32) for _ in range(_NBUF)]
        + [pltpu.SemaphoreType.DMA for _ in range(2 * _NBUF)]
    )

    @functools.partial(
        pl.kernel,
        out_type=jax.ShapeDtypeStruct((B, D), jnp.float32),
        mesh=mesh,
        scratch_types=scratch,
        compiler_params=pltpu.CompilerParams(use_tc_tiling_on_sc=False),
    )
    def gather_kernel(idx_hbm, table_hbm, out_hbm, idx_v, *rest):
        rows = rest[:_NBUF]
        gsem = rest[_NBUF:2 * _NBUF]
        wsem = rest[2 * _NBUF:]
        wid = lax.axis_index("s") * _NC + lax.axis_index("c")
        base = wid * b_per_w

        pltpu.sync_copy(idx_hbm.at[wid], idx_v)

        for b in range(_NBUF):
            pltpu.async_copy(table_hbm.at[idx_v.at[b]], rows[b], gsem[b])

        @pl.loop(0, nchunks, step=_NBUF)
        def _(g):
            for b in range(_NBUF):
                j = g + b
                pltpu.make_async_copy(
                    table_hbm.at[idx_v.at[j]], rows[b], gsem[b]).wait()
                dst = out_hbm.at[pl.ds(base + j * _CHUNK, _CHUNK)]
                pltpu.async_copy(rows[b], dst, wsem[b])

                @pl.when(j + _NBUF < nchunks)
                def _():
                    pltpu.make_async_copy(rows[b], dst, wsem[b]).wait()
                    pltpu.async_copy(
                        table_hbm.at[idx_v.at[j + _NBUF]], rows[b], gsem[b])

        for b in range(_NBUF):
            j = nchunks - _NBUF + b
            pltpu.make_async_copy(
                rows[b],
                out_hbm.at[pl.ds(base + j * _CHUNK, _CHUNK)],
                wsem[b]).wait()

    return gather_kernel


def kernel(text, embedding_weight):
    Bt, L = text.shape
    V, D = embedding_weight.shape
    B = Bt * L
    idx = text.reshape(_NW, (B // _NW) // _CHUNK, _CHUNK).astype(jnp.int32)
    out = _make_gather(B, D)(idx, embedding_weight)
    return out.reshape(Bt, 1, L, D)
